# baseline (speedup 1.0000x reference)
import jax
import jax.numpy as jnp
from jax import lax
from jax.experimental import pallas as pl
from jax.experimental.pallas import tpu as pltpu
from jax.experimental.pallas import tpu_sc as plsc

BATCH = 16384
NUM_CONT = 13
NUM_ENUM = 26
VOCAB = 100000
EMB = 16
OUT_W = NUM_CONT + NUM_ENUM * EMB

NC = 2
NS = 16
NW = NC * NS
ROWS_PER_W = BATCH // NW
CHUNK = 64
N_CHUNK = ROWS_PER_W // CHUNK

NTILE = VOCAB // 128
VTAIL = VOCAB - NTILE * 128
S_PER_W = (NTILE + NW - 1) // NW


def _relayout_body(tp_hbm, tail_hbm, t2_hbm, buf, trow, sem_in, sem_out):
    wid = lax.axis_index("s") * NC + lax.axis_index("c")
    s_lo = wid * S_PER_W
    n_s = jnp.minimum(NTILE, s_lo + S_PER_W) - s_lo
    k_tot = NUM_ENUM * n_s

    def in_copy(k, p):
        i = k // n_s
        s = s_lo + k % n_s
        off = pl.multiple_of(s * 128, 128)
        return pltpu.make_async_copy(
            tp_hbm.at[i].at[:, pl.ds(off, 128)],
            buf.at[p].at[:, pl.ds(0, 128)], sem_in)

    def out_copy(k, p):
        i = k // n_s
        s = s_lo + k % n_s
        off = pl.multiple_of((i * VOCAB + s * 128) * EMB, 2048)
        return pltpu.make_async_copy(
            trow.at[p], t2_hbm.at[pl.ds(off, 128 * EMB)], sem_out)

    NBUF = 4

    def run():
        for j in range(NBUF - 1):
            in_copy(j, j).start()

        def step(k, carry):
            p = k % NBUF

            @pl.when(k + NBUF - 1 < k_tot)
            def _():
                in_copy(k + NBUF - 1, (k + NBUF - 1) % NBUF).start()

            in_copy(k, p).wait()

            @pl.when(k >= NBUF)
            def _():
                out_copy(k - NBUF, p).wait()

            rows16 = lax.iota(jnp.int32, 16)
            cols = jnp.zeros((16,), jnp.int32)
            for g in range(16):
                vals = []
                for m in range(8):
                    vals.append(plsc.load_gather(buf.at[p], [rows16, cols]))
                    cols = cols + 1
                for m in range(8):
                    trow[p, pl.ds((g * 8 + m) * 16, 16)] = vals[m]

            out_copy(k, p).start()
            return carry

        lax.fori_loop(0, k_tot, step, 0)

        for j in range(NBUF):
            kk = k_tot - NBUF + j
            out_copy(kk, kk % NBUF).wait()

    run()

    @pl.when(wid < NUM_ENUM)
    def _():
        pltpu.sync_copy(
            tail_hbm.at[pl.ds(pl.multiple_of(wid * VTAIL * EMB, 512),
                              VTAIL * EMB)],
            t2_hbm.at[pl.ds(pl.multiple_of(
                (wid * VOCAB + NTILE * 128) * EMB, 512), VTAIL * EMB)])


def _lookup_body(x_hbm, xet_hbm, tab_hbm, out_hbm, idxbuf, xbuf, gbuf, outv, sem):
    wid = lax.axis_index("s") * NC + lax.axis_index("c")
    base = wid * ROWS_PER_W

    pltpu.sync_copy(xet_hbm.at[:, pl.ds(pl.multiple_of(base, ROWS_PER_W),
                                        ROWS_PER_W)], idxbuf)

    def chunk_body(c, carry):
        row0 = base + c * CHUNK
        pltpu.sync_copy(
            x_hbm.at[pl.ds(pl.multiple_of(row0 * NUM_CONT, CHUNK * NUM_CONT),
                           CHUNK * NUM_CONT)],
            xbuf.at[pl.ds(0, CHUNK * NUM_CONT)])

        copies = [
            pltpu.make_async_copy(
                tab_hbm.at[i].at[idxbuf.at[i, pl.ds(c * CHUNK, CHUNK)]],
                gbuf.at[pl.ds(i * CHUNK, CHUNK)], sem)
            for i in range(NUM_ENUM)
        ]
        for cp in copies:
            cp.start()
        for cp in copies:
            cp.wait()

        def row_body(r, rc):
            outv[r, pl.ds(0, 16)] = xbuf[pl.ds(r * NUM_CONT, 16)]
            for i in range(NUM_ENUM):
                outv[r, pl.ds(NUM_CONT + i * EMB, EMB)] = gbuf[i * CHUNK + r, :]
            return rc

        lax.fori_loop(0, CHUNK, row_body, 0)
        pltpu.sync_copy(outv,
                        out_hbm.at[pl.ds(pl.multiple_of(row0, CHUNK), CHUNK)])
        return carry

    lax.fori_loop(0, N_CHUNK, chunk_body, 0)


@jax.jit
def kernel(x, xe, tables):
    mesh = plsc.VectorSubcoreMesh(core_axis_name="c", subcore_axis_name="s")
    relayout = pl.kernel(
        _relayout_body, mesh=mesh,
        out_type=jax.ShapeDtypeStruct((NUM_ENUM * VOCAB * EMB,), jnp.float32),
        scratch_types=[
            pltpu.VMEM((4, EMB, 129), jnp.float32),
            pltpu.VMEM((4, 128 * EMB), jnp.float32),
            pltpu.SemaphoreType.DMA,
            pltpu.SemaphoreType.DMA,
        ],
        compiler_params=pltpu.CompilerParams(use_tc_tiling_on_sc=True,
                                             needs_layout_passes=False),
    )
    lookup = pl.kernel(
        _lookup_body, mesh=mesh,
        out_type=jax.ShapeDtypeStruct((BATCH, OUT_W), jnp.float32),
        scratch_types=[
            pltpu.VMEM((NUM_ENUM, ROWS_PER_W), jnp.int32),
            pltpu.VMEM((CHUNK * NUM_CONT + 16,), jnp.float32),
            pltpu.VMEM((NUM_ENUM * CHUNK, EMB), jnp.float32),
            pltpu.VMEM((CHUNK, OUT_W), jnp.float32),
            pltpu.SemaphoreType.DMA,
        ],
        compiler_params=pltpu.CompilerParams(use_tc_tiling_on_sc=False),
    )
    tp = jnp.transpose(tables, (0, 2, 1))
    tail = tables[:, NTILE * 128:, :].reshape(-1)
    t2 = relayout(tp, tail)
    return lookup(x.reshape(-1), xe.T, t2.reshape(NUM_ENUM, VOCAB, EMB))


# speedup vs baseline: 1.3410x; 1.3410x over previous
import jax
import jax.numpy as jnp
from jax import lax
from jax.experimental import pallas as pl
from jax.experimental.pallas import tpu as pltpu
from jax.experimental.pallas import tpu_sc as plsc

BATCH = 16384
NUM_CONT = 13
NUM_ENUM = 26
VOCAB = 100000
EMB = 16
OUT_W = NUM_CONT + NUM_ENUM * EMB

NC = 2
NS = 16
NW = NC * NS
ROWS_PER_W = BATCH // NW
CHUNK = 64
N_CHUNK = ROWS_PER_W // CHUNK

NTILE = VOCAB // 128
VTAIL = VOCAB - NTILE * 128
S_PER_W = (NTILE + NW - 1) // NW


def _relayout_body(tp_hbm, tail_hbm, t2_hbm, buf, trow, sem_in, sem_out):
    wid = lax.axis_index("s") * NC + lax.axis_index("c")
    s_lo = wid * S_PER_W
    n_s = jnp.minimum(NTILE, s_lo + S_PER_W) - s_lo
    k_tot = NUM_ENUM * n_s

    def in_copy(k, p):
        i = k // n_s
        s = s_lo + k % n_s
        off = pl.multiple_of(s * 128, 128)
        return pltpu.make_async_copy(
            tp_hbm.at[i].at[:, pl.ds(off, 128)],
            buf.at[pl.ds(p * EMB, EMB), pl.ds(0, 128)], sem_in)

    def out_copy(k, p):
        i = k // n_s
        s = s_lo + k % n_s
        off = pl.multiple_of((i * VOCAB + s * 128) * EMB, 2048)
        return pltpu.make_async_copy(
            trow.at[pl.ds(p * 128 * EMB, 128 * EMB)],
            t2_hbm.at[pl.ds(off, 128 * EMB)], sem_out)

    NBUF = 4

    def run():
        for j in range(NBUF - 1):
            in_copy(j, j).start()

        def step(k, carry):
            p = k % NBUF

            @pl.when(k + NBUF - 1 < k_tot)
            def _():
                in_copy(k + NBUF - 1, (k + NBUF - 1) % NBUF).start()

            in_copy(k, p).wait()

            @pl.when(k >= NBUF)
            def _():
                out_copy(k - NBUF, p).wait()

            idx0 = lax.iota(jnp.int32, 16) * EMB
            for e in range(EMB):
                for g in range(8):
                    vals = buf[p * EMB + e, pl.ds(g * 16, 16)]
                    plsc.store_scatter(
                        trow,
                        [idx0 + (p * 128 * EMB + g * 16 * EMB + e)], vals)

            out_copy(k, p).start()
            return carry

        lax.fori_loop(0, k_tot, step, 0)

        for j in range(NBUF):
            kk = k_tot - NBUF + j
            out_copy(kk, kk % NBUF).wait()

    run()

    @pl.when(wid < NUM_ENUM)
    def _():
        pltpu.sync_copy(
            tail_hbm.at[pl.ds(pl.multiple_of(wid * VTAIL * EMB, 512),
                              VTAIL * EMB)],
            t2_hbm.at[pl.ds(pl.multiple_of(
                (wid * VOCAB + NTILE * 128) * EMB, 512), VTAIL * EMB)])


def _lookup_body(x_hbm, xet_hbm, tab_hbm, out_hbm, idxbuf, xbuf, gbuf, outv, sem):
    wid = lax.axis_index("s") * NC + lax.axis_index("c")
    base = wid * ROWS_PER_W

    pltpu.sync_copy(xet_hbm.at[:, pl.ds(pl.multiple_of(base, ROWS_PER_W),
                                        ROWS_PER_W)], idxbuf)

    def chunk_body(c, carry):
        row0 = base + c * CHUNK
        pltpu.sync_copy(
            x_hbm.at[pl.ds(pl.multiple_of(row0 * NUM_CONT, CHUNK * NUM_CONT),
                           CHUNK * NUM_CONT)],
            xbuf.at[pl.ds(0, CHUNK * NUM_CONT)])

        copies = [
            pltpu.make_async_copy(
                tab_hbm.at[i].at[idxbuf.at[i, pl.ds(c * CHUNK, CHUNK)]],
                gbuf.at[pl.ds(i * CHUNK, CHUNK)], sem)
            for i in range(NUM_ENUM)
        ]
        for cp in copies:
            cp.start()
        for cp in copies:
            cp.wait()

        def row_body(r, rc):
            outv[r, pl.ds(0, 16)] = xbuf[pl.ds(r * NUM_CONT, 16)]
            for i in range(NUM_ENUM):
                outv[r, pl.ds(NUM_CONT + i * EMB, EMB)] = gbuf[i * CHUNK + r, :]
            return rc

        lax.fori_loop(0, CHUNK, row_body, 0)
        pltpu.sync_copy(outv,
                        out_hbm.at[pl.ds(pl.multiple_of(row0, CHUNK), CHUNK)])
        return carry

    lax.fori_loop(0, N_CHUNK, chunk_body, 0)


@jax.jit
def kernel(x, xe, tables):
    mesh = plsc.VectorSubcoreMesh(core_axis_name="c", subcore_axis_name="s")
    relayout = pl.kernel(
        _relayout_body, mesh=mesh,
        out_type=jax.ShapeDtypeStruct((NUM_ENUM * VOCAB * EMB,), jnp.float32),
        scratch_types=[
            pltpu.VMEM((4 * EMB, 129), jnp.float32),
            pltpu.VMEM((4 * 128 * EMB,), jnp.float32),
            pltpu.SemaphoreType.DMA,
            pltpu.SemaphoreType.DMA,
        ],
        compiler_params=pltpu.CompilerParams(use_tc_tiling_on_sc=True,
                                             needs_layout_passes=False),
    )
    lookup = pl.kernel(
        _lookup_body, mesh=mesh,
        out_type=jax.ShapeDtypeStruct((BATCH, OUT_W), jnp.float32),
        scratch_types=[
            pltpu.VMEM((NUM_ENUM, ROWS_PER_W), jnp.int32),
            pltpu.VMEM((CHUNK * NUM_CONT + 16,), jnp.float32),
            pltpu.VMEM((NUM_ENUM * CHUNK, EMB), jnp.float32),
            pltpu.VMEM((CHUNK, OUT_W), jnp.float32),
            pltpu.SemaphoreType.DMA,
        ],
        compiler_params=pltpu.CompilerParams(use_tc_tiling_on_sc=False),
    )
    tp = jnp.transpose(tables, (0, 2, 1))
    tail = tables[:, NTILE * 128:, :].reshape(-1)
    t2 = relayout(tp, tail)
    return lookup(x.reshape(-1), xe.T, t2.reshape(NUM_ENUM, VOCAB, EMB))
